# baseline (speedup 1.0000x reference)
import jax
import jax.numpy as jnp
from jax import lax
from jax.experimental import pallas as pl
from jax.experimental.pallas import tpu as pltpu
from jax.experimental.pallas import tpu_sc as plsc

_NUM_CORES = 2
_NUM_SUBCORES = 16
_NUM_WORKERS = _NUM_CORES * _NUM_SUBCORES

_PACK = 64
_BLK_ROWS = 256


def _score_body(w_ref, tbl_ref, out_ref):
    x = tbl_ref[...]
    y = jnp.dot(x, w_ref[...], preferred_element_type=jnp.float32)
    out_ref[:, 0:_PACK] = y


def _compute_scores(object_table):
    v, d = object_table.shape
    tblf = object_table.reshape(v // _PACK, _PACK * d)
    char = lax.slice(tblf, (0, 0), (1, d))[0]
    eye = jnp.eye(_PACK, dtype=jnp.float32)
    w = (eye[:, None, :] * char[None, :, None]).reshape(_PACK * d, _PACK)
    nrows = v // _PACK
    nblk = -(-nrows // _BLK_ROWS)
    out = pl.pallas_call(
        _score_body,
        grid=(nblk,),
        in_specs=[
            pl.BlockSpec((_PACK * d, _PACK), lambda i: (0, 0)),
            pl.BlockSpec((_BLK_ROWS, _PACK * d), lambda i: (i, 0)),
        ],
        out_specs=pl.BlockSpec((_BLK_ROWS, 2 * _PACK), lambda i: (i, 0)),
        out_shape=jax.ShapeDtypeStruct((nblk * _BLK_ROWS, 2 * _PACK), jnp.float32),
    )(w, tblf)
    return out.reshape(nblk * _BLK_ROWS * 2 * _PACK)


def _gather_body(per_w, scores_hbm, idx_hbm, out_hbm, idx_v, out_v, sem):
    wid = lax.axis_index("s") * _NUM_CORES + lax.axis_index("c")
    base = wid * per_w
    pltpu.sync_copy(idx_hbm.at[pl.ds(base, per_w)], idx_v)

    def remap(i, _):
        a = idx_v[pl.ds(i * 16, 16)]
        idx_v[pl.ds(i * 16, 16)] = (a << 1) - (a & 63)
        return _

    lax.fori_loop(0, per_w // 16, remap, 0)
    pltpu.async_copy(scores_hbm.at[idx_v], out_v, sem).wait()
    pltpu.sync_copy(out_v, out_hbm.at[pl.ds(base, per_w)])


def _gather_scores(scores, idx_flat):
    n = idx_flat.shape[0]
    per_w = n // _NUM_WORKERS
    mesh = plsc.VectorSubcoreMesh(
        core_axis_name="c", subcore_axis_name="s",
        num_cores=_NUM_CORES, num_subcores=_NUM_SUBCORES)

    def body(scores_hbm, idx_hbm, out_hbm, idx_v, out_v, sem):
        _gather_body(per_w, scores_hbm, idx_hbm, out_hbm, idx_v, out_v, sem)

    f = pl.kernel(
        body,
        mesh=mesh,
        out_type=jax.ShapeDtypeStruct((n,), jnp.float32),
        scratch_types=[
            pltpu.VMEM((per_w,), jnp.int32),
            pltpu.VMEM((per_w,), jnp.float32),
            pltpu.SemaphoreType.DMA,
        ],
    )
    return f(scores, idx_flat)


def kernel(indices, object_table):
    b, l = indices.shape
    scores = _compute_scores(object_table)
    out = _gather_scores(scores, indices.reshape(-1))
    return out.reshape(b, l)


# speedup vs baseline: 1.1565x; 1.1565x over previous
import jax
import jax.numpy as jnp
from jax import lax
from jax.experimental import pallas as pl
from jax.experimental.pallas import tpu as pltpu
from jax.experimental.pallas import tpu_sc as plsc

_NUM_CORES = 2
_NUM_SUBCORES = 16
_NUM_WORKERS = _NUM_CORES * _NUM_SUBCORES

_STRIP = 4096
_PACK8 = 8


def _score_body(c_ref, tbl_ref, out_ref):
    i = pl.program_id(0)
    x = tbl_ref[...]
    s = lax.dot_general(c_ref[...], x, (((1,), (1,)), ((), ())),
                        preferred_element_type=jnp.float32)
    out_ref[pl.ds(lax.rem(i, _PACK8), 1), :] = s


def _compute_scores(object_table):
    v, d = object_table.shape
    c2d = lax.slice(object_table, (0, 0), (1, d))
    nblk = -(-v // _STRIP)
    nout = -(-nblk // _PACK8)
    out = pl.pallas_call(
        _score_body,
        grid=(nblk,),
        in_specs=[
            pl.BlockSpec((1, d), lambda i: (0, 0)),
            pl.BlockSpec((_STRIP, d), lambda i: (i, 0)),
        ],
        out_specs=pl.BlockSpec((_PACK8, _STRIP), lambda i: (i // _PACK8, 0)),
        out_shape=jax.ShapeDtypeStruct((nout * _PACK8, _STRIP), jnp.float32),
    )(c2d, object_table)
    return out.reshape(nout * _PACK8 * _STRIP)


def _gather_body(per_w, scores_hbm, idx_hbm, out_hbm, idx_v, out_v, sem):
    wid = lax.axis_index("s") * _NUM_CORES + lax.axis_index("c")
    base = wid * per_w
    pltpu.sync_copy(idx_hbm.at[pl.ds(base, per_w)], idx_v)
    pltpu.async_copy(scores_hbm.at[idx_v], out_v, sem).wait()
    pltpu.sync_copy(out_v, out_hbm.at[pl.ds(base, per_w)])


def _gather_scores(scores, idx_flat):
    n = idx_flat.shape[0]
    per_w = n // _NUM_WORKERS
    mesh = plsc.VectorSubcoreMesh(
        core_axis_name="c", subcore_axis_name="s",
        num_cores=_NUM_CORES, num_subcores=_NUM_SUBCORES)

    def body(scores_hbm, idx_hbm, out_hbm, idx_v, out_v, sem):
        _gather_body(per_w, scores_hbm, idx_hbm, out_hbm, idx_v, out_v, sem)

    f = pl.kernel(
        body,
        mesh=mesh,
        out_type=jax.ShapeDtypeStruct((n,), jnp.float32),
        scratch_types=[
            pltpu.VMEM((per_w,), jnp.int32),
            pltpu.VMEM((per_w,), jnp.float32),
            pltpu.SemaphoreType.DMA,
        ],
    )
    return f(scores, idx_flat)


def kernel(indices, object_table):
    b, l = indices.shape
    scores = _compute_scores(object_table)
    out = _gather_scores(scores, indices.reshape(-1))
    return out.reshape(b, l)


# speedup vs baseline: 1.3562x; 1.1727x over previous
import jax
import jax.numpy as jnp
from jax import lax
from jax.experimental import pallas as pl
from jax.experimental.pallas import tpu as pltpu
from jax.experimental.pallas import tpu_sc as plsc

_NUM_CORES = 2
_NUM_SUBCORES = 16
_NUM_WORKERS = _NUM_CORES * _NUM_SUBCORES

_STRIP = 16384
_PACK8 = 8


def _score_body(c_ref, tbl_ref, out_ref):
    i = pl.program_id(0)
    x = tbl_ref[...]
    s = lax.dot_general(c_ref[...], x, (((1,), (1,)), ((), ())),
                        preferred_element_type=jnp.float32)
    out_ref[pl.ds(lax.rem(i, _PACK8), 1), :] = s


def _compute_scores(object_table):
    v, d = object_table.shape
    c2d = lax.slice(object_table, (0, 0), (1, d))
    nblk = -(-v // _STRIP)
    nout = -(-nblk // _PACK8)
    out = pl.pallas_call(
        _score_body,
        grid=(nblk,),
        in_specs=[
            pl.BlockSpec((1, d), lambda i: (0, 0)),
            pl.BlockSpec((_STRIP, d), lambda i: (i, 0)),
        ],
        out_specs=pl.BlockSpec((_PACK8, _STRIP), lambda i: (i // _PACK8, 0)),
        out_shape=jax.ShapeDtypeStruct((nout * _PACK8, _STRIP), jnp.float32),
    )(c2d, object_table)
    return out.reshape(nout * _PACK8 * _STRIP)


def _gather_body(per_w, scores_hbm, idx_hbm, out_hbm, idx_v, out_v, sem):
    wid = lax.axis_index("s") * _NUM_CORES + lax.axis_index("c")
    base = wid * per_w
    pltpu.sync_copy(idx_hbm.at[pl.ds(base, per_w)], idx_v)
    pltpu.async_copy(scores_hbm.at[idx_v], out_v, sem).wait()
    pltpu.sync_copy(out_v, out_hbm.at[pl.ds(base, per_w)])


def _gather_scores(scores, idx_flat):
    n = idx_flat.shape[0]
    per_w = n // _NUM_WORKERS
    mesh = plsc.VectorSubcoreMesh(
        core_axis_name="c", subcore_axis_name="s",
        num_cores=_NUM_CORES, num_subcores=_NUM_SUBCORES)

    def body(scores_hbm, idx_hbm, out_hbm, idx_v, out_v, sem):
        _gather_body(per_w, scores_hbm, idx_hbm, out_hbm, idx_v, out_v, sem)

    f = pl.kernel(
        body,
        mesh=mesh,
        out_type=jax.ShapeDtypeStruct((n,), jnp.float32),
        scratch_types=[
            pltpu.VMEM((per_w,), jnp.int32),
            pltpu.VMEM((per_w,), jnp.float32),
            pltpu.SemaphoreType.DMA,
        ],
    )
    return f(scores, idx_flat)


def kernel(indices, object_table):
    b, l = indices.shape
    scores = _compute_scores(object_table)
    out = _gather_scores(scores, indices.reshape(-1))
    return out.reshape(b, l)
